# baseline (speedup 1.0000x reference)
import functools

import jax
import jax.numpy as jnp
from jax import lax
from jax.experimental import pallas as pl
from jax.experimental.pallas import tpu as pltpu
from jax.experimental.pallas import tpu_sc as plsc

_SCALE = 65536.0
_NSEG = 16
_SLOPE_MASK = -8388608
_B_MASK = 0x1FFFF

_EYTZ_PERM = (8, 4, 9, 2, 10, 5, 11, 1, 12, 6, 13, 3, 14, 7, 15)

_TC_ELEMS = 10485760


def _pack_tables(intercepts, signs, exps):
    sign_bit = ((1 - signs) // 2).astype(jnp.int32)
    slope_bits = (sign_bit << 31) | ((127 + exps) << 23)
    return slope_bits | (intercepts + 65536)


def _thresholds(breakpoints):
    bpf = breakpoints.astype(jnp.float32)
    pos = bpf / _SCALE
    neg = jnp.nextafter(bpf - 1.0, jnp.float32(jnp.inf)) / _SCALE
    return jnp.where(breakpoints > 0, pos, neg)


def _eytz_tree(thresholds):
    return jnp.zeros((16,), jnp.float32).at[jnp.array(_EYTZ_PERM)].set(thresholds)


def _tc_body(t_ref, packed_ref, x_ref, o_ref):
    x = x_ref[...]
    acc = jnp.where(x >= t_ref[0], packed_ref[1], packed_ref[0])
    for j in range(1, _NSEG - 1):
        acc = jnp.where(x >= t_ref[j], packed_ref[j + 1], acc)
    slope = lax.bitcast_convert_type(acc & _SLOPE_MASK, jnp.float32)
    b = (acc & _B_MASK).astype(jnp.float32) * (1.0 / _SCALE) - 1.0
    o_ref[...] = b + slope * x


_TC_BLK = 2097152
_TC_COLS = 128


def _tc_call(x2, thresholds, packed, write_elems, out_rows):
    br = _TC_BLK // _TC_COLS
    grid = write_elems // _TC_BLK
    return pl.pallas_call(
        _tc_body,
        grid=(grid,),
        in_specs=[
            pl.BlockSpec(memory_space=pltpu.SMEM),
            pl.BlockSpec(memory_space=pltpu.SMEM),
            pl.BlockSpec((br, _TC_COLS), lambda i: (i, 0)),
        ],
        out_specs=pl.BlockSpec((br, _TC_COLS), lambda i: (i, 0)),
        out_shape=jax.ShapeDtypeStruct((out_rows, _TC_COLS), jnp.float32),
    )(thresholds, packed, x2)


_SC_CH = 16384


def _sc_table(thresholds, intercepts, signs, exps):
    b = intercepts.astype(jnp.float32) / _SCALE
    s = signs.astype(jnp.float32) * jnp.exp2(exps.astype(jnp.float32))
    z = jnp.zeros((64,), jnp.float32)
    return z.at[jnp.array(_EYTZ_PERM)].set(thresholds).at[32:48].set(b).at[48:64].set(s)


def _sc_call(x, comb, skip=0):
    n = x.shape[0]
    nw = 32
    per_w = (n - skip) // nw
    nch = per_w // _SC_CH
    assert (n - skip) % nw == 0 and per_w % _SC_CH == 0 and nch % 2 == 0
    mesh = plsc.VectorSubcoreMesh(core_axis_name="c", subcore_axis_name="s")

    @functools.partial(
        pl.kernel,
        out_type=jax.ShapeDtypeStruct((n - skip,), jnp.float32),
        mesh=mesh,
        compiler_params=pltpu.CompilerParams(needs_layout_passes=False),
        scratch_types=[
            pltpu.VMEM((64,), jnp.float32),
            pltpu.VMEM((_SC_CH,), jnp.float32),
            pltpu.VMEM((_SC_CH,), jnp.float32),
            pltpu.VMEM((_SC_CH,), jnp.float32),
            pltpu.VMEM((_SC_CH,), jnp.float32),
            pltpu.SemaphoreType.DMA,
            pltpu.SemaphoreType.DMA,
            pltpu.SemaphoreType.DMA,
            pltpu.SemaphoreType.DMA,
        ],
    )
    def sck(x_hbm, tab_hbm, o_hbm, tab_v,
            in0, in1, out0, out1, si0, si1, so0, so1):
        wid = lax.axis_index("s") * 2 + lax.axis_index("c")
        rbase = skip + wid * per_w
        base = wid * per_w
        pltpu.sync_copy(tab_hbm, tab_v)
        ins = (in0, in1)
        outs = (out0, out1)
        sis = (si0, si1)
        sos = (so0, so1)

        def start_in(g, s):
            pltpu.async_copy(
                x_hbm.at[pl.ds(rbase + g * _SC_CH, _SC_CH)], ins[s], sis[s])

        def start_out(g, s):
            pltpu.async_copy(
                outs[s], o_hbm.at[pl.ds(base + g * _SC_CH, _SC_CH)], sos[s])

        def wait_in(s):
            pltpu.make_async_copy(
                x_hbm.at[pl.ds(rbase, _SC_CH)], ins[s], sis[s]).wait()

        def wait_out(s):
            pltpu.make_async_copy(
                outs[s], o_hbm.at[pl.ds(base, _SC_CH)], sos[s]).wait()

        def compute(s):
            xin = ins[s]
            yout = outs[s]

            def vbody(o2):
                xv = xin[pl.ds(o2, 16)]
                node = jnp.ones((16,), jnp.int32)
                for _ in range(4):
                    tv = plsc.load_gather(tab_v, [node])
                    node = node + node + (xv >= tv).astype(jnp.int32)
                b = plsc.load_gather(tab_v, [node + 16])
                s = plsc.load_gather(tab_v, [node + 32])
                yout[pl.ds(o2, 16)] = b + s * xv

            plsc.parallel_loop(0, _SC_CH, 16, unroll=8)(vbody)

        start_in(0, 0)
        start_in(1, 1)

        def pair_body(g2, c):
            for s in (0, 1):
                g = 2 * g2 + s
                wait_in(s)

                @pl.when(g2 >= 1)
                def _():
                    wait_out(s)

                compute(s)

                @pl.when(g2 < nch // 2 - 1)
                def _():
                    start_in(g + 2, s)

                start_out(g, s)
            return c

        lax.fori_loop(0, nch // 2, pair_body, 0)
        wait_out(0)
        wait_out(1)

    return sck(x, comb)


def kernel(x, breakpoints, intercepts, signs, exps):
    packed = _pack_tables(intercepts, signs, exps)
    thr = _thresholds(breakpoints)
    comb = _sc_table(thr, intercepts, signs, exps)
    n = x.shape[0]
    cols = _TC_COLS
    x2 = x.reshape(n // cols, cols)
    k = min(_TC_ELEMS, n) if n % _TC_BLK == 0 else 0
    if k == 0:
        return _sc_call(x, comb).astype(x.dtype)
    if k == n:
        return _tc_call(x2, thr, packed, n, n // cols).reshape(n).astype(x.dtype)
    sc_out = _sc_call(x, comb, skip=k)
    tc_full = _tc_call(x2, thr, packed, k, n // cols)
    out = lax.dynamic_update_slice(
        tc_full, sc_out.reshape((n - k) // cols, cols), (k // cols, 0))
    return out.reshape(n).astype(x.dtype)


# speedup vs baseline: 1.0253x; 1.0253x over previous
import functools

import jax
import jax.numpy as jnp
from jax import lax
from jax.experimental import pallas as pl
from jax.experimental.pallas import tpu as pltpu
from jax.experimental.pallas import tpu_sc as plsc

_SCALE = 65536.0
_NSEG = 16
_SLOPE_MASK = -8388608
_B_MASK = 0x1FFFF

_EYTZ_PERM = (8, 4, 9, 2, 10, 5, 11, 1, 12, 6, 13, 3, 14, 7, 15)

_TC_ELEMS = 10485760


def _pack_tables(intercepts, signs, exps):
    sign_bit = ((1 - signs) // 2).astype(jnp.int32)
    slope_bits = (sign_bit << 31) | ((127 + exps) << 23)
    return slope_bits | (intercepts + 65536)


def _thresholds(breakpoints):
    bpf = breakpoints.astype(jnp.float32)
    pos = bpf / _SCALE
    neg = jnp.nextafter(bpf - 1.0, jnp.float32(jnp.inf)) / _SCALE
    return jnp.where(breakpoints > 0, pos, neg)


def _tc_body(t_ref, packed_ref, x_ref, o_ref):
    x = x_ref[...]
    acc = jnp.where(x >= t_ref[0], packed_ref[1], packed_ref[0])
    for j in range(1, _NSEG - 1):
        acc = jnp.where(x >= t_ref[j], packed_ref[j + 1], acc)
    slope = lax.bitcast_convert_type(acc & _SLOPE_MASK, jnp.float32)
    b = (acc & _B_MASK).astype(jnp.float32) * (1.0 / _SCALE) - 1.0
    o_ref[...] = b + slope * x


_TC_BLK = 1048576
_TC_COLS = 128


def _tc_call(x2, thresholds, packed, write_elems, out_rows):
    br = _TC_BLK // _TC_COLS
    grid = write_elems // _TC_BLK
    return pl.pallas_call(
        _tc_body,
        grid=(grid,),
        in_specs=[
            pl.BlockSpec(memory_space=pltpu.SMEM),
            pl.BlockSpec(memory_space=pltpu.SMEM),
            pl.BlockSpec((br, _TC_COLS), lambda i: (i, 0)),
        ],
        out_specs=pl.BlockSpec((br, _TC_COLS), lambda i: (i, 0)),
        out_shape=jax.ShapeDtypeStruct((out_rows, _TC_COLS), jnp.float32),
    )(thresholds, packed, x2)


_SC_CH = 16384


def _sc_table(thresholds, intercepts, signs, exps):
    b = intercepts.astype(jnp.float32) / _SCALE
    s = signs.astype(jnp.float32) * jnp.exp2(exps.astype(jnp.float32))
    z = jnp.zeros((64,), jnp.float32)
    return z.at[jnp.array(_EYTZ_PERM)].set(thresholds).at[32:48].set(b).at[48:64].set(s)


def _sc_call(x, comb, skip=0):
    n = x.shape[0]
    nw = 32
    per_w = (n - skip) // nw
    nch = per_w // _SC_CH
    assert (n - skip) % nw == 0 and per_w % _SC_CH == 0 and nch % 2 == 0
    mesh = plsc.VectorSubcoreMesh(core_axis_name="c", subcore_axis_name="s")

    @functools.partial(
        pl.kernel,
        out_type=jax.ShapeDtypeStruct((n - skip,), jnp.float32),
        mesh=mesh,
        compiler_params=pltpu.CompilerParams(needs_layout_passes=False),
        scratch_types=[
            pltpu.VMEM((64,), jnp.float32),
            pltpu.VMEM((_SC_CH,), jnp.float32),
            pltpu.VMEM((_SC_CH,), jnp.float32),
            pltpu.VMEM((_SC_CH,), jnp.float32),
            pltpu.VMEM((_SC_CH,), jnp.float32),
            pltpu.SemaphoreType.DMA,
            pltpu.SemaphoreType.DMA,
            pltpu.SemaphoreType.DMA,
            pltpu.SemaphoreType.DMA,
        ],
    )
    def sck(x_hbm, tab_hbm, o_hbm, tab_v,
            in0, in1, out0, out1, si0, si1, so0, so1):
        wid = lax.axis_index("s") * 2 + lax.axis_index("c")
        rbase = skip + wid * per_w
        base = wid * per_w
        pltpu.sync_copy(tab_hbm, tab_v)
        ins = (in0, in1)
        outs = (out0, out1)
        sis = (si0, si1)
        sos = (so0, so1)

        def start_in(g, s):
            pltpu.async_copy(
                x_hbm.at[pl.ds(rbase + g * _SC_CH, _SC_CH)], ins[s], sis[s])

        def start_out(g, s):
            pltpu.async_copy(
                outs[s], o_hbm.at[pl.ds(base + g * _SC_CH, _SC_CH)], sos[s])

        def wait_in(s):
            pltpu.make_async_copy(
                x_hbm.at[pl.ds(rbase, _SC_CH)], ins[s], sis[s]).wait()

        def wait_out(s):
            pltpu.make_async_copy(
                outs[s], o_hbm.at[pl.ds(base, _SC_CH)], sos[s]).wait()

        def compute(s):
            xin = ins[s]
            yout = outs[s]

            def vbody(o2):
                xv = xin[pl.ds(o2, 16)]
                node = jnp.ones((16,), jnp.int32)
                for _ in range(4):
                    tv = plsc.load_gather(tab_v, [node])
                    node = node + node + (xv >= tv).astype(jnp.int32)
                b = plsc.load_gather(tab_v, [node + 16])
                s = plsc.load_gather(tab_v, [node + 32])
                yout[pl.ds(o2, 16)] = b + s * xv

            plsc.parallel_loop(0, _SC_CH, 16, unroll=8)(vbody)

        start_in(0, 0)
        start_in(1, 1)

        def pair_body(g2, c):
            for s in (0, 1):
                g = 2 * g2 + s
                wait_in(s)

                @pl.when(g2 >= 1)
                def _():
                    wait_out(s)

                compute(s)

                @pl.when(g2 < nch // 2 - 1)
                def _():
                    start_in(g + 2, s)

                start_out(g, s)
            return c

        lax.fori_loop(0, nch // 2, pair_body, 0)
        wait_out(0)
        wait_out(1)

    return sck(x, comb)


def kernel(x, breakpoints, intercepts, signs, exps):
    packed = _pack_tables(intercepts, signs, exps)
    thr = _thresholds(breakpoints)
    comb = _sc_table(thr, intercepts, signs, exps)
    n = x.shape[0]
    cols = _TC_COLS
    x2 = x.reshape(n // cols, cols)
    k = min(_TC_ELEMS, n) if n % _TC_BLK == 0 else 0
    if k == 0:
        return _sc_call(x, comb).astype(x.dtype)
    if k == n:
        return _tc_call(x2, thr, packed, n, n // cols).reshape(n).astype(x.dtype)
    sc_out = _sc_call(x, comb, skip=k)
    tc_full = _tc_call(x2, thr, packed, k, n // cols)
    out = lax.dynamic_update_slice(
        tc_full, sc_out.reshape((n - k) // cols, cols), (k // cols, 0))
    return out.reshape(n).astype(x.dtype)
